# baseline (speedup 1.0000x reference)
import functools

import jax
import jax.numpy as jnp
from jax import lax
from jax.experimental import pallas as pl
from jax.experimental.pallas import tpu as pltpu
from jax.experimental.pallas import tpu_sc as plsc

_N = 10000
_E = 160000
_D = 256
_DH = 128
_NC = 2
_NS = 16
_CH = 128
_NCHP = 1280
_EP = _NCHP * _CH
_NROWS = 10016
_NBUF = 2
_PCH = 40
_STRIPE = 640
_LAST = _N - (_NS - 1) * _STRIPE
_ROWB = 1000
_NB = _N // _ROWB
_NP = 10112
_DSTRIPE = 640
_DLAST = _NP - (_NS - 1) * _DSTRIPE

_f32 = jnp.float32


@functools.lru_cache(maxsize=None)
def _sc_mesh():
    return plsc.VectorSubcoreMesh(
        core_axis_name="c", subcore_axis_name="s",
        num_cores=_NC, num_subcores=_NS)


def _copy_stripes(src_sh, dst_hbm, s):
    @pl.when(s < _NS - 1)
    def _():
        pltpu.sync_copy(src_sh.at[pl.ds(s * _STRIPE, _STRIPE)],
                        dst_hbm.at[pl.ds(s * _STRIPE, _STRIPE)])

    @pl.when(s == _NS - 1)
    def _():
        pltpu.sync_copy(src_sh.at[pl.ds((_NS - 1) * _STRIPE, _LAST)],
                        dst_hbm.at[pl.ds((_NS - 1) * _STRIPE, _LAST)])


def _zero_acc_stripes(z_hbm, acc_sh, s):
    @pl.when(s < _NS - 1)
    def _():
        pltpu.sync_copy(z_hbm, acc_sh.at[pl.ds(s * _STRIPE, _STRIPE)])

    @pl.when(s == _NS - 1)
    def _():
        pltpu.sync_copy(z_hbm.at[pl.ds(0, _LAST)],
                        acc_sh.at[pl.ds((_NS - 1) * _STRIPE, _LAST)])


_DCPW = _NCHP // (_NC * _NS)


def _deg_body(s0, d0, s1, d1, zeros_n, ones_c, out,
              idx_v, ones_v, sh0, sh1, sh2, sh3, sem):
    c = lax.axis_index("c")
    s = lax.axis_index("s")
    wid = c * _NS + s

    @pl.when(s == 0)
    def _():
        pltpu.sync_copy(zeros_n, sh0)
        pltpu.sync_copy(zeros_n, sh1)
        pltpu.sync_copy(zeros_n, sh2)
        pltpu.sync_copy(zeros_n, sh3)

    pltpu.sync_copy(ones_c, ones_v)
    plsc.subcore_barrier()

    for ei, sh in ((s0, sh0), (d0, sh1), (s1, sh2), (d1, sh3)):
        pltpu.sync_copy(ei.at[pl.ds(wid * _DCPW, _DCPW)], idx_v)

        def grp(g, _, sh=sh):
            descs = []
            for b in range(10):
                descs.append(pltpu.async_copy(
                    ones_v, sh.at[idx_v.at[g * 10 + b]], sem, add=True))
            for d in descs:
                d.wait()
            return 0

        lax.fori_loop(0, _DCPW // 10, grp, 0)

    plsc.subcore_barrier()
    for a, sh in enumerate((sh0, sh1, sh2, sh3)):
        @pl.when(s < _NS - 1)
        def _(sh=sh, a=a):
            pltpu.sync_copy(sh.at[pl.ds(s * _DSTRIPE, _DSTRIPE)],
                            out.at[c, a, pl.ds(s * _DSTRIPE, _DSTRIPE)])

        @pl.when(s == _NS - 1)
        def _(sh=sh, a=a):
            pltpu.sync_copy(sh.at[pl.ds((_NS - 1) * _DSTRIPE, _DLAST)],
                            out.at[c, a, pl.ds((_NS - 1) * _DSTRIPE, _DLAST)])


@functools.lru_cache(maxsize=None)
def _deg_call():
    return pl.kernel(
        _deg_body,
        out_type=jax.ShapeDtypeStruct((_NC, 4, _NP), _f32),
        mesh=_sc_mesh(),
        scratch_types=[
            pltpu.VMEM((_DCPW, _CH), jnp.int32),
            pltpu.VMEM((_CH,), _f32),
            pltpu.VMEM_SHARED((_NP,), _f32),
            pltpu.VMEM_SHARED((_NP,), _f32),
            pltpu.VMEM_SHARED((_NP,), _f32),
            pltpu.VMEM_SHARED((_NP,), _f32),
            pltpu.SemaphoreType.DMA,
        ],
    )


def _scale_body(degp_ref, so_ref, si_ref):
    deg = degp_ref[0] + degp_ref[1]
    sc = lax.rsqrt(jnp.maximum(deg, 1.0))
    so_ref[...] = jnp.stack([sc[0, :_N], sc[2, :_N]], axis=1)
    si_ref[...] = jnp.stack([sc[1, :_N], sc[3, :_N]], axis=1)


_scale_call = pl.pallas_call(
    _scale_body,
    out_shape=(jax.ShapeDtypeStruct((_N, 2), _f32),
               jax.ShapeDtypeStruct((_N, 2), _f32)),
)


def _mm0_body(x_ref, so_ref, w0_ref, w1_ref, p0a, p0b, p1a, p1b):
    xb = x_ref[...]
    for r, (w_ref, pa, pb) in enumerate(((w0_ref, p0a, p0b),
                                         (w1_ref, p1a, p1b))):
        h = xb * so_ref[:, r:r + 1]
        p = jnp.dot(h, w_ref[...], preferred_element_type=_f32)
        pa[...] = p[:, :_DH]
        pb[...] = p[:, _DH:]


_mm0_call = pl.pallas_call(
    _mm0_body,
    grid=(_NB,),
    in_specs=[
        pl.BlockSpec((_ROWB, _D), lambda j: (j, 0)),
        pl.BlockSpec((_ROWB, 2), lambda j: (j, 0)),
        pl.BlockSpec((_D, _D), lambda j: (0, 0)),
        pl.BlockSpec((_D, _D), lambda j: (0, 0)),
    ],
    out_specs=[pl.BlockSpec((_ROWB, _DH), lambda j: (j, 0))] * 4,
    out_shape=[jax.ShapeDtypeStruct((_NROWS, _DH), _f32)] * 4,
)


def _mm_mid_body(split, a0a, a0b, a1a, a1b, si_ref, b_ref, so_ref,
                 w0_ref, w1_ref, *outs):
    si0 = si_ref[:, 0:1]
    si1 = si_ref[:, 1:2]
    ha = si0 * a0a[...] + si1 * a1a[...] + b_ref[0, :_DH]
    hb = si0 * a0b[...] + si1 * a1b[...] + b_ref[0, _DH:]
    h = jnp.concatenate([ha, hb], axis=1)
    for r, w_ref in enumerate((w0_ref, w1_ref)):
        p = jnp.dot(h * so_ref[:, r:r + 1], w_ref[...],
                    preferred_element_type=_f32)
        if split:
            outs[2 * r][...] = p[:, :_DH]
            outs[2 * r + 1][...] = p[:, _DH:]
        else:
            outs[r][...] = p


def _make_mm_mid(split, dout):
    n_out = 4 if split else 2
    ob = dout // 2 if split else dout
    return pl.pallas_call(
        functools.partial(_mm_mid_body, split),
        grid=(_NB,),
        in_specs=[
            pl.BlockSpec((_ROWB, _DH), lambda j: (j, 0)),
            pl.BlockSpec((_ROWB, _DH), lambda j: (j, 0)),
            pl.BlockSpec((_ROWB, _DH), lambda j: (j, 0)),
            pl.BlockSpec((_ROWB, _DH), lambda j: (j, 0)),
            pl.BlockSpec((_ROWB, 2), lambda j: (j, 0)),
            pl.BlockSpec((1, _D), lambda j: (0, 0)),
            pl.BlockSpec((_ROWB, 2), lambda j: (j, 0)),
            pl.BlockSpec((_D, dout), lambda j: (0, 0)),
            pl.BlockSpec((_D, dout), lambda j: (0, 0)),
        ],
        out_specs=[pl.BlockSpec((_ROWB, ob), lambda j: (j, 0))] * n_out,
        out_shape=[jax.ShapeDtypeStruct((_NROWS, ob), _f32)] * n_out,
    )


_mm_mid_call = _make_mm_mid(True, _D)
_mm_last_call = _make_mm_mid(False, _DH)


def _fin_body(p0A, p0B, p1A, p1B, si_ref, b_ref, out_ref):
    si0 = si_ref[:, 0:1]
    si1 = si_ref[:, 1:2]
    out_ref[...] = (si0 * (p0A[...] + p0B[...])
                    + si1 * (p1A[...] + p1B[...]) + b_ref[0, :])


_fin_call = pl.pallas_call(
    _fin_body,
    grid=(_NB,),
    in_specs=[
        pl.BlockSpec((_ROWB, _DH), lambda j: (j, 0)),
        pl.BlockSpec((_ROWB, _DH), lambda j: (j, 0)),
        pl.BlockSpec((_ROWB, _DH), lambda j: (j, 0)),
        pl.BlockSpec((_ROWB, _DH), lambda j: (j, 0)),
        pl.BlockSpec((_ROWB, 2), lambda j: (j, 0)),
        pl.BlockSpec((1, _DH), lambda j: (0, 0)),
    ],
    out_specs=pl.BlockSpec((_ROWB, _DH), lambda j: (j, 0)),
    out_shape=jax.ShapeDtypeStruct((_N, _DH), _f32),
)


def _prop_pipeline(tab, acc, src, dst, base, cpw,
                   bufs, srcv, dstv, fsems, gsems, ssems):
    def start_f(c, j):
        pltpu.async_copy(src.at[base + c], srcv.at[j], fsems[j])
        pltpu.async_copy(dst.at[base + c], dstv.at[j], fsems[j])

    def wait_f(c, j):
        pltpu.make_async_copy(src.at[base + c], srcv.at[j], fsems[j]).wait()
        pltpu.make_async_copy(dst.at[base + c], dstv.at[j], fsems[j]).wait()

    def start_g(b, j):
        pltpu.async_copy(tab.at[srcv.at[j]], bufs[b], gsems[b])

    def wait_g(b, j):
        pltpu.make_async_copy(tab.at[srcv.at[j]], bufs[b], gsems[b]).wait()

    def start_s(b, j):
        pltpu.async_copy(bufs[b], acc.at[dstv.at[j]], ssems[b], add=True)

    def wait_s(b, j):
        pltpu.make_async_copy(bufs[b], acc.at[dstv.at[j]], ssems[b]).wait()

    for j in range(3):
        start_f(j, j)
    wait_f(0, 0)
    start_g(0, 0)

    def step(k, _):
        for u in range(4):
            c = 4 * k + u
            wait_g(u % 2, u)
            start_s(u % 2, u)

            @pl.when(c + 1 < cpw)
            def _(c=c, j1=(u + 1) % 4):
                wait_f(c + 1, j1)

            @pl.when(c >= 1)
            def _(b1=(u + 1) % 2, j3=(u + 3) % 4):
                wait_s(b1, j3)

            @pl.when(c + 3 < cpw)
            def _(c=c, j3=(u + 3) % 4):
                start_f(c + 3, j3)

            @pl.when(c + 1 < cpw)
            def _(b1=(u + 1) % 2, j1=(u + 1) % 4):
                start_g(b1, j1)

        return 0

    lax.fori_loop(0, cpw // 4, step, 0)
    wait_s((cpw - 1) % 2, (cpw - 1) % 4)


_CPW256 = _NCHP // _NS


def _prop256_body(ta0, tb0, ta1, tb1, s0e, d0e, s1e, d1e, z2,
                  o0a, o0b, o1a, o1b,
                  srcv, dstv, b0, b1, acc,
                  f0, f1, f2, f3, g0, g1, s0, s1):
    c = lax.axis_index("c")
    s = lax.axis_index("s")
    rings = ((b0, b1), srcv, dstv, (f0, f1, f2, f3), (g0, g1), (s0, s1))

    def one_rel(tab, src, dst, out):
        _zero_acc_stripes(z2, acc, s)
        plsc.subcore_barrier()
        _prop_pipeline(tab, acc, src, dst, s * _CPW256, _CPW256, *rings)
        plsc.subcore_barrier()
        _copy_stripes(acc, out, s)

    @pl.when(c == 0)
    def _():
        one_rel(ta0, s0e, d0e, o0a)
        one_rel(ta1, s1e, d1e, o1a)

    @pl.when(c == 1)
    def _():
        one_rel(tb0, s0e, d0e, o0b)
        one_rel(tb1, s1e, d1e, o1b)


def _prop_scratch():
    return ([
        pltpu.VMEM((4, _CH), jnp.int32),
        pltpu.VMEM((4, _CH), jnp.int32),
    ] + [pltpu.VMEM((_CH, _DH), _f32)] * _NBUF
      + [pltpu.VMEM_SHARED((_NROWS, _DH), _f32)]
      + [pltpu.SemaphoreType.DMA] * (4 + 2 * _NBUF))


@functools.lru_cache(maxsize=None)
def _prop256_call():
    return pl.kernel(
        _prop256_body,
        out_type=tuple(jax.ShapeDtypeStruct((_N, _DH), _f32)
                       for _ in range(4)),
        mesh=_sc_mesh(),
        scratch_types=_prop_scratch(),
    )


_CPW128 = _NCHP // (_NC * _NS)


def _prop128_body(t0, t1, s0e, d0e, s1e, d1e, z2,
                  o0a, o0b, o1a, o1b,
                  srcv, dstv, b0, b1, acc,
                  f0, f1, f2, f3, g0, g1, s0, s1):
    c = lax.axis_index("c")
    s = lax.axis_index("s")
    wid = c * _NS + s
    rings = ((b0, b1), srcv, dstv, (f0, f1, f2, f3), (g0, g1), (s0, s1))

    def one_rel(tab, src, dst, out_a, out_b):
        _zero_acc_stripes(z2, acc, s)
        plsc.subcore_barrier()
        _prop_pipeline(tab, acc, src, dst, wid * _CPW128, _CPW128, *rings)
        plsc.subcore_barrier()

        @pl.when(c == 0)
        def _():
            _copy_stripes(acc, out_a, s)

        @pl.when(c == 1)
        def _():
            _copy_stripes(acc, out_b, s)

    one_rel(t0, s0e, d0e, o0a, o0b)
    one_rel(t1, s1e, d1e, o1a, o1b)


@functools.lru_cache(maxsize=None)
def _prop128_call():
    return pl.kernel(
        _prop128_body,
        out_type=tuple(jax.ShapeDtypeStruct((_N, _DH), _f32)
                       for _ in range(4)),
        mesh=_sc_mesh(),
        scratch_types=_prop_scratch(),
    )


def kernel(x, edge_index_rel0, edge_index_rel1,
           W0_r0, b0_r0, W0_r1, b0_r1,
           W1_r0, b1_r0, W1_r1, b1_r1,
           W2_r0, b2_r0, W2_r1, b2_r1):
    pad = _N + (jnp.arange(_EP - _E, dtype=jnp.int32) % (_NROWS - _N))

    def _padr(a):
        return jnp.concatenate([a, pad]).reshape(_NCHP, _CH)

    s0 = _padr(edge_index_rel0[0])
    d0 = _padr(edge_index_rel0[1])
    s1 = _padr(edge_index_rel1[0])
    d1 = _padr(edge_index_rel1[1])

    zeros_n = jnp.zeros((_NP,), _f32)
    ones_c = jnp.ones((_CH,), _f32)
    z2 = jnp.zeros((_STRIPE, _DH), _f32)

    degp = _deg_call()(s0, d0, s1, d1, zeros_n, ones_c)
    so, si = _scale_call(degp)

    b0 = (b0_r0 + b0_r1).reshape(1, _D)
    b1 = (b1_r0 + b1_r1).reshape(1, _D)
    b2 = (b2_r0 + b2_r1).reshape(1, _DH)

    prop256 = _prop256_call()
    prop128 = _prop128_call()

    p0a, p0b, p1a, p1b = _mm0_call(x, so, W0_r0, W0_r1)
    a0a, a0b, a1a, a1b = prop256(p0a, p0b, p1a, p1b, s0, d0, s1, d1, z2)

    q0a, q0b, q1a, q1b = _mm_mid_call(a0a, a0b, a1a, a1b, si, b0, so,
                                      W1_r0, W1_r1)
    a0a, a0b, a1a, a1b = prop256(q0a, q0b, q1a, q1b, s0, d0, s1, d1, z2)

    p0, p1 = _mm_last_call(a0a, a0b, a1a, a1b, si, b1, so, W2_r0, W2_r1)
    r0A, r0B, r1A, r1B = prop128(p0, p1, s0, d0, s1, d1, z2)

    return _fin_call(r0A, r0B, r1A, r1B, si, b2)
